# baseline (speedup 1.0000x reference)
import functools

import jax
import jax.numpy as jnp
from jax import lax
from jax.experimental import pallas as pl
from jax.experimental.pallas import tpu as pltpu
from jax.experimental.pallas import tpu_sc as plsc

B, CIN, HID, LAT, K = 4096, 768, 1024, 32, 512
TILE = 1024
GRID = B // TILE
NCAND = 3
GLANE = 128


def _dot(a, b, precision=None):
    return jax.lax.dot_general(
        a, b, (((1,), (0,)), ((), ())),
        preferred_element_type=jnp.float32,
        precision=precision)


def _encode_kernel(x_ref, embt_ref, w1_ref, b1_ref, w2_ref, b2_ref,
                   w3_ref, b3_ref, f_ref, idx_ref):
    h1 = jax.nn.relu(_dot(x_ref[...], w1_ref[...]) + b1_ref[...])
    h2 = jax.nn.relu(_dot(h1, w2_ref[...]) + b2_ref[...])
    f = jax.nn.relu(_dot(h2, w3_ref[...]) + b3_ref[...])
    f_ref[...] = f

    embt = embt_ref[...]
    esq = jnp.sum(embt * embt, axis=0, keepdims=True)
    sc = _dot(f, embt, precision=jax.lax.Precision.HIGHEST)
    score = esq - (sc + sc)

    iota = jax.lax.broadcasted_iota(jnp.int32, (TILE, K), 1)
    s_cur = score
    for c in range(NCAND):
        m = jnp.min(s_cur, axis=1, keepdims=True)
        i_c = jnp.min(jnp.where(s_cur <= m, iota, K), axis=1)[:, None]
        idx_ref[:, c:c + 1] = i_c
        s_cur = jnp.where(iota == i_c, jnp.inf, s_cur)


def _sc_gather(emb_hbm, idx_hbm, out_hbm, idx_v, rows_v, sem):
    info = plsc.get_sparse_core_info()
    nw = info.num_cores * info.num_subcores
    b_per_w = (NCAND * B) // nw
    wid = lax.axis_index("s") * info.num_cores + lax.axis_index("c")
    base = wid * b_per_w
    pltpu.sync_copy(idx_hbm.at[pl.ds(base, b_per_w)], idx_v)
    pltpu.async_copy(emb_hbm.at[idx_v], rows_v, sem).wait()
    pltpu.sync_copy(rows_v, out_hbm.at[pl.ds(base, b_per_w)])


def _decode_kernel(f_ref, idx_ref, cand_ref, w4_ref, b4_ref, w5_ref, b5_ref,
                   w6_ref, b6_ref, recon_ref, q_ref, loss_ref):
    i = pl.program_id(0)
    f = f_ref[...]

    def exact_dist(ec):
        t = f - ec
        t = t * t
        acc = None
        for a in range(4):
            v = t[:, 8 * a:8 * a + 8]
            v = v[:, 0:4] + v[:, 4:8]
            v = v[:, 0:2] + v[:, 2:4]
            v = v[:, 0:1] + v[:, 1:2]
            acc = v if acc is None else acc + v
        return acc

    d_w = i_w = e_w = None
    for c in range(NCAND):
        e_c = cand_ref[pl.ds(c * TILE, TILE), :LAT]
        i_c = idx_ref[:, c:c + 1]
        d_c = exact_dist(e_c)
        if c == 0:
            d_w, i_w, e_w = d_c, i_c, e_c
        else:
            take = (d_c < d_w) | ((d_c == d_w) & (i_c < i_w))
            d_w = jnp.where(take, d_c, d_w)
            i_w = jnp.where(take, i_c, i_w)
            e_w = jnp.where(take, e_c, e_w)

    q = e_w
    dq = q - f
    part = jnp.sum(dq * dq)

    @pl.when(i == 0)
    def _():
        loss_ref[0, 0] = 0.0
    loss_ref[0, 0] += part

    q = f + dq
    d1 = jax.nn.relu(_dot(q, w4_ref[...]) + b4_ref[...])
    d2 = jax.nn.relu(_dot(d1, w5_ref[...]) + b5_ref[...])
    recon_ref[...] = jax.nn.sigmoid(_dot(d2, w6_ref[...]) + b6_ref[...])
    q_ref[...] = q


@jax.jit
def kernel(x, emb, enc1_w, enc1_b, enc2_w, enc2_b, enc3_w, enc3_b,
           dec1_w, dec1_b, dec2_w, dec2_b, dec3_w, dec3_b):
    rep = lambda shape: pl.BlockSpec(shape, lambda i: (0,) * len(shape))

    f, idx = pl.pallas_call(
        _encode_kernel,
        grid=(GRID,),
        in_specs=[pl.BlockSpec((TILE, CIN), lambda i: (i, 0)),
                  rep((LAT, K)),
                  rep((CIN, HID)), rep((1, HID)),
                  rep((HID, HID)), rep((1, HID)),
                  rep((HID, LAT)), rep((1, LAT))],
        out_specs=[pl.BlockSpec((TILE, LAT), lambda i: (i, 0)),
                   pl.BlockSpec((TILE, NCAND), lambda i: (i, 0))],
        out_shape=[jax.ShapeDtypeStruct((B, LAT), jnp.float32),
                   jax.ShapeDtypeStruct((B, NCAND), jnp.int32)],
    )(x, emb.T, enc1_w.T, enc1_b.reshape(1, -1), enc2_w.T,
      enc2_b.reshape(1, -1), enc3_w.T, enc3_b.reshape(1, -1))

    idx_flat = idx.reshape(GRID, TILE, NCAND).transpose(0, 2, 1).reshape(NCAND * B)

    info = plsc.get_sparse_core_info()
    nw = info.num_cores * info.num_subcores
    b_per_w = (NCAND * B) // nw
    cand = functools.partial(
        pl.kernel, mesh=plsc.VectorSubcoreMesh(core_axis_name="c",
                                               subcore_axis_name="s"),
        out_type=jax.ShapeDtypeStruct((NCAND * B, GLANE), jnp.float32),
        scratch_types=[pltpu.VMEM((b_per_w,), jnp.int32),
                       pltpu.VMEM((b_per_w, GLANE), jnp.float32),
                       pltpu.SemaphoreType.DMA],
    )(_sc_gather)(jnp.pad(emb, ((0, 0), (0, GLANE - LAT))), idx_flat)

    recon, q, loss = pl.pallas_call(
        _decode_kernel,
        grid=(GRID,),
        in_specs=[pl.BlockSpec((TILE, LAT), lambda i: (i, 0)),
                  pl.BlockSpec((TILE, NCAND), lambda i: (i, 0)),
                  pl.BlockSpec((NCAND * TILE, GLANE), lambda i: (i, 0)),
                  rep((LAT, HID)), rep((1, HID)),
                  rep((HID, HID)), rep((1, HID)),
                  rep((HID, CIN)), rep((1, CIN))],
        out_specs=[pl.BlockSpec((TILE, CIN), lambda i: (i, 0)),
                   pl.BlockSpec((TILE, LAT), lambda i: (i, 0)),
                   pl.BlockSpec(memory_space=pltpu.SMEM)],
        out_shape=[jax.ShapeDtypeStruct((B, CIN), jnp.float32),
                   jax.ShapeDtypeStruct((B, LAT), jnp.float32),
                   jax.ShapeDtypeStruct((1, 1), jnp.float32)],
    )(f, idx, cand,
      dec1_w.T, dec1_b.reshape(1, -1), dec2_w.T, dec2_b.reshape(1, -1),
      dec3_w.T, dec3_b.reshape(1, -1))

    vq_loss = loss[0, 0] * (1.25 / (B * LAT))
    return (recon, vq_loss, q)


# speedup vs baseline: 1.9875x; 1.9875x over previous
import functools

import jax
import jax.numpy as jnp
from jax.experimental import pallas as pl
from jax.experimental.pallas import tpu as pltpu

B, CIN, HID, LAT, K = 4096, 768, 1024, 32, 512
TILE = 512
GRID = B // TILE


def _dot(a, b, precision=None):
    return jax.lax.dot_general(
        a, b, (((1,), (0,)), ((), ())),
        preferred_element_type=jnp.float32,
        precision=precision)


def _vqvae_kernel(x_ref, emb_ref, embt_ref,
                  w1_ref, b1_ref, w2_ref, b2_ref, w3_ref, b3_ref,
                  w4_ref, b4_ref, w5_ref, b5_ref, w6_ref, b6_ref,
                  recon_ref, q_ref, loss_ref):
    i = pl.program_id(0)

    h1 = jax.nn.relu(_dot(x_ref[...], w1_ref[...]) + b1_ref[...])
    h2 = jax.nn.relu(_dot(h1, w2_ref[...]) + b2_ref[...])
    f = jax.nn.relu(_dot(h2, w3_ref[...]) + b3_ref[...])

    embt = embt_ref[...]
    esq = jnp.sum(embt * embt, axis=0, keepdims=True)
    sc = _dot(f, embt, precision=jax.lax.Precision.HIGHEST)
    score = esq - (sc + sc)

    iota = jax.lax.broadcasted_iota(jnp.int32, (TILE, K), 1)

    def first_min_idx(s):
        m = jnp.min(s, axis=1, keepdims=True)
        return jnp.min(jnp.where(s <= m, iota, K), axis=1)[:, None]

    def exact_dist(ec):
        t = f - ec
        t = t * t
        acc = None
        for a in range(4):
            v = t[:, 8 * a:8 * a + 8]
            v = v[:, 0:4] + v[:, 4:8]
            v = v[:, 0:2] + v[:, 2:4]
            v = v[:, 0:1] + v[:, 1:2]
            acc = v if acc is None else acc + v
        return acc

    cands = []
    s_cur = score
    for _ in range(3):
        i_c = first_min_idx(s_cur)
        oh = (iota == i_c).astype(jnp.float32)
        e_c = _dot(oh, emb_ref[...], precision=jax.lax.Precision.HIGHEST)
        cands.append((exact_dist(e_c), i_c, e_c))
        s_cur = jnp.where(iota == i_c, jnp.inf, s_cur)

    d_w, i_w, e_w = cands[0]
    for d_c, i_c, e_c in cands[1:]:
        take = (d_c < d_w) | ((d_c == d_w) & (i_c < i_w))
        d_w = jnp.where(take, d_c, d_w)
        i_w = jnp.where(take, i_c, i_w)
        e_w = jnp.where(take, e_c, e_w)
    q = e_w

    dq = q - f
    part = jnp.sum(dq * dq)
    q = f + dq

    @pl.when(i == 0)
    def _():
        loss_ref[0, 0] = 0.0
    loss_ref[0, 0] += part

    d1 = jax.nn.relu(_dot(q, w4_ref[...]) + b4_ref[...])
    d2 = jax.nn.relu(_dot(d1, w5_ref[...]) + b5_ref[...])
    recon_ref[...] = jax.nn.sigmoid(_dot(d2, w6_ref[...]) + b6_ref[...])
    q_ref[...] = q


@functools.partial(jax.jit, static_argnames=("interpret",))
def kernel(x, emb, enc1_w, enc1_b, enc2_w, enc2_b, enc3_w, enc3_b,
           dec1_w, dec1_b, dec2_w, dec2_b, dec3_w, dec3_b, interpret=False):
    rep = lambda shape: pl.BlockSpec(shape, lambda i: (0,) * len(shape))
    w_specs = []
    ws = []
    for w, b in ((enc1_w, enc1_b), (enc2_w, enc2_b), (enc3_w, enc3_b),
                 (dec1_w, dec1_b), (dec2_w, dec2_b), (dec3_w, dec3_b)):
        wt = w.T
        ws += [wt, b.reshape(1, -1)]
        w_specs += [rep(wt.shape), rep((1, b.shape[0]))]

    recon, q, loss = pl.pallas_call(
        _vqvae_kernel,
        grid=(GRID,),
        in_specs=[pl.BlockSpec((TILE, CIN), lambda i: (i, 0)),
                  rep((K, LAT)), rep((LAT, K))] + w_specs,
        out_specs=[pl.BlockSpec((TILE, CIN), lambda i: (i, 0)),
                   pl.BlockSpec((TILE, LAT), lambda i: (i, 0)),
                   pl.BlockSpec(memory_space=pltpu.SMEM)],
        out_shape=[jax.ShapeDtypeStruct((B, CIN), jnp.float32),
                   jax.ShapeDtypeStruct((B, LAT), jnp.float32),
                   jax.ShapeDtypeStruct((1, 1), jnp.float32)],
        interpret=interpret,
    )(x, emb, emb.T, *ws)

    vq_loss = loss[0, 0] * (1.25 / (B * LAT))
    return (recon, vq_loss, q)


# speedup vs baseline: 2.0412x; 1.0271x over previous
import functools

import jax
import jax.numpy as jnp
from jax.experimental import pallas as pl
from jax.experimental.pallas import tpu as pltpu

B, CIN, HID, LAT, K = 4096, 768, 1024, 32, 512
TILE = 1024
GRID = B // TILE


def _dot(a, b, precision=None):
    return jax.lax.dot_general(
        a, b, (((1,), (0,)), ((), ())),
        preferred_element_type=jnp.float32,
        precision=precision)


def _vqvae_kernel(x_ref, emb_ref, embt_ref,
                  w1_ref, b1_ref, w2_ref, b2_ref, w3_ref, b3_ref,
                  w4_ref, b4_ref, w5_ref, b5_ref, w6_ref, b6_ref,
                  recon_ref, q_ref, loss_ref):
    i = pl.program_id(0)

    h1 = jax.nn.relu(_dot(x_ref[...], w1_ref[...]) + b1_ref[...])
    h2 = jax.nn.relu(_dot(h1, w2_ref[...]) + b2_ref[...])
    f = jax.nn.relu(_dot(h2, w3_ref[...]) + b3_ref[...])

    embt = embt_ref[...]
    esq = jnp.sum(embt * embt, axis=0, keepdims=True)
    sc = _dot(f, embt, precision=jax.lax.Precision.HIGHEST)
    score = esq - (sc + sc)

    iota = jax.lax.broadcasted_iota(jnp.int32, (TILE, K), 1)

    def first_min_idx(s):
        m = jnp.min(s, axis=1, keepdims=True)
        return jnp.min(jnp.where(s <= m, iota, K), axis=1)[:, None]

    def exact_dist(ec):
        t = f - ec
        t = t * t
        acc = None
        for a in range(4):
            v = t[:, 8 * a:8 * a + 8]
            v = v[:, 0:4] + v[:, 4:8]
            v = v[:, 0:2] + v[:, 2:4]
            v = v[:, 0:1] + v[:, 1:2]
            acc = v if acc is None else acc + v
        return acc

    cands = []
    s_cur = score
    for _ in range(3):
        i_c = first_min_idx(s_cur)
        oh = (iota == i_c).astype(jnp.float32)
        e_c = _dot(oh, emb_ref[...], precision=jax.lax.Precision.HIGHEST)
        cands.append((exact_dist(e_c), i_c, e_c))
        s_cur = jnp.where(iota == i_c, jnp.inf, s_cur)

    d_w, i_w, e_w = cands[0]
    for d_c, i_c, e_c in cands[1:]:
        take = (d_c < d_w) | ((d_c == d_w) & (i_c < i_w))
        d_w = jnp.where(take, d_c, d_w)
        i_w = jnp.where(take, i_c, i_w)
        e_w = jnp.where(take, e_c, e_w)
    q = e_w

    dq = q - f
    part = jnp.sum(dq * dq)
    q = f + dq

    @pl.when(i == 0)
    def _():
        loss_ref[0, 0] = 0.0
    loss_ref[0, 0] += part

    d1 = jax.nn.relu(_dot(q, w4_ref[...]) + b4_ref[...])
    d2 = jax.nn.relu(_dot(d1, w5_ref[...]) + b5_ref[...])
    recon_ref[...] = jax.nn.sigmoid(_dot(d2, w6_ref[...]) + b6_ref[...])
    q_ref[...] = q


@functools.partial(jax.jit, static_argnames=("interpret",))
def kernel(x, emb, enc1_w, enc1_b, enc2_w, enc2_b, enc3_w, enc3_b,
           dec1_w, dec1_b, dec2_w, dec2_b, dec3_w, dec3_b, interpret=False):
    rep = lambda shape: pl.BlockSpec(shape, lambda i: (0,) * len(shape))
    w_specs = []
    ws = []
    for w, b in ((enc1_w, enc1_b), (enc2_w, enc2_b), (enc3_w, enc3_b),
                 (dec1_w, dec1_b), (dec2_w, dec2_b), (dec3_w, dec3_b)):
        wt = w.T
        ws += [wt, b.reshape(1, -1)]
        w_specs += [rep(wt.shape), rep((1, b.shape[0]))]

    recon, q, loss = pl.pallas_call(
        _vqvae_kernel,
        grid=(GRID,),
        in_specs=[pl.BlockSpec((TILE, CIN), lambda i: (i, 0)),
                  rep((K, LAT)), rep((LAT, K))] + w_specs,
        out_specs=[pl.BlockSpec((TILE, CIN), lambda i: (i, 0)),
                   pl.BlockSpec((TILE, LAT), lambda i: (i, 0)),
                   pl.BlockSpec(memory_space=pltpu.SMEM)],
        out_shape=[jax.ShapeDtypeStruct((B, CIN), jnp.float32),
                   jax.ShapeDtypeStruct((B, LAT), jnp.float32),
                   jax.ShapeDtypeStruct((1, 1), jnp.float32)],
        interpret=interpret,
    )(x, emb, emb.T, *ws)

    vq_loss = loss[0, 0] * (1.25 / (B * LAT))
    return (recon, vq_loss, q)


# speedup vs baseline: 2.3697x; 1.1609x over previous
import functools

import jax
import jax.numpy as jnp
from jax.experimental import pallas as pl
from jax.experimental.pallas import tpu as pltpu

B, CIN, HID, LAT, K = 4096, 768, 1024, 32, 512
TILE = 1024
GRID = B // TILE


def _dot(a, b, precision=None):
    return jax.lax.dot_general(
        a, b, (((1,), (0,)), ((), ())),
        preferred_element_type=jnp.float32,
        precision=precision)


def _dott(a, w, precision=None):
    return jax.lax.dot_general(
        a, w, (((1,), (1,)), ((), ())),
        preferred_element_type=jnp.float32,
        precision=precision)


def _vqvae_kernel(x_ref, emb_ref, embt_ref,
                  w1_ref, b1_ref, w2_ref, b2_ref, w3_ref, b3_ref,
                  w4_ref, b4_ref, w5_ref, b5_ref, w6_ref, b6_ref,
                  recon_ref, q_ref, loss_ref):
    i = pl.program_id(0)

    h1 = jax.nn.relu(_dott(x_ref[...], w1_ref[...]) + b1_ref[...])
    h2 = jax.nn.relu(_dott(h1, w2_ref[...]) + b2_ref[...])
    f = jax.nn.relu(_dott(h2, w3_ref[...]) + b3_ref[...])

    embt = embt_ref[...]
    esq = jnp.sum(embt * embt, axis=0, keepdims=True)
    sc = _dot(f, embt, precision=jax.lax.Precision.HIGHEST)
    score = esq - (sc + sc)

    iota = jax.lax.broadcasted_iota(jnp.int32, (TILE, K), 1)

    def first_min_idx(s):
        m = jnp.min(s, axis=1, keepdims=True)
        return jnp.min(jnp.where(s <= m, iota, K), axis=1)[:, None]

    def exact_dist(ec):
        t = f - ec
        t = t * t
        acc = None
        for a in range(4):
            v = t[:, 8 * a:8 * a + 8]
            v = v[:, 0:4] + v[:, 4:8]
            v = v[:, 0:2] + v[:, 2:4]
            v = v[:, 0:1] + v[:, 1:2]
            acc = v if acc is None else acc + v
        return acc

    cands = []
    s_cur = score
    for _ in range(3):
        i_c = first_min_idx(s_cur)
        oh = (iota == i_c).astype(jnp.float32)
        e_c = _dot(oh, emb_ref[...], precision=jax.lax.Precision.HIGHEST)
        cands.append((exact_dist(e_c), i_c, e_c))
        s_cur = jnp.where(iota == i_c, jnp.inf, s_cur)

    d_w, i_w, e_w = cands[0]
    for d_c, i_c, e_c in cands[1:]:
        take = (d_c < d_w) | ((d_c == d_w) & (i_c < i_w))
        d_w = jnp.where(take, d_c, d_w)
        i_w = jnp.where(take, i_c, i_w)
        e_w = jnp.where(take, e_c, e_w)
    q = e_w

    dq = q - f
    part = jnp.sum(dq * dq)
    q = f + dq

    @pl.when(i == 0)
    def _():
        loss_ref[0, 0] = 0.0
    loss_ref[0, 0] += part

    d1 = jax.nn.relu(_dott(q, w4_ref[...]) + b4_ref[...])
    d2 = jax.nn.relu(_dott(d1, w5_ref[...]) + b5_ref[...])
    recon_ref[...] = jax.nn.sigmoid(_dott(d2, w6_ref[...]) + b6_ref[...])
    q_ref[...] = q


@functools.partial(jax.jit, static_argnames=("interpret",))
def kernel(x, emb, enc1_w, enc1_b, enc2_w, enc2_b, enc3_w, enc3_b,
           dec1_w, dec1_b, dec2_w, dec2_b, dec3_w, dec3_b, interpret=False):
    rep = lambda shape: pl.BlockSpec(shape, lambda i: (0,) * len(shape))
    w_specs = []
    ws = []
    for w, b in ((enc1_w, enc1_b), (enc2_w, enc2_b), (enc3_w, enc3_b),
                 (dec1_w, dec1_b), (dec2_w, dec2_b), (dec3_w, dec3_b)):
        ws += [w, b.reshape(1, -1)]
        w_specs += [rep(w.shape), rep((1, b.shape[0]))]

    recon, q, loss = pl.pallas_call(
        _vqvae_kernel,
        grid=(GRID,),
        in_specs=[pl.BlockSpec((TILE, CIN), lambda i: (i, 0)),
                  rep((K, LAT)), rep((LAT, K))] + w_specs,
        out_specs=[pl.BlockSpec((TILE, CIN), lambda i: (i, 0)),
                   pl.BlockSpec((TILE, LAT), lambda i: (i, 0)),
                   pl.BlockSpec(memory_space=pltpu.SMEM)],
        out_shape=[jax.ShapeDtypeStruct((B, CIN), jnp.float32),
                   jax.ShapeDtypeStruct((B, LAT), jnp.float32),
                   jax.ShapeDtypeStruct((1, 1), jnp.float32)],
        interpret=interpret,
    )(x, emb, emb.T, *ws)

    vq_loss = loss[0, 0] * (1.25 / (B * LAT))
    return (recon, vq_loss, q)


# speedup vs baseline: 2.3966x; 1.0113x over previous
import functools

import jax
import jax.numpy as jnp
from jax.experimental import pallas as pl
from jax.experimental.pallas import tpu as pltpu

B, CIN, HID, LAT, K = 4096, 768, 1024, 32, 512
TILE = 1024
GRID = B // TILE


def _dot(a, b, precision=None):
    return jax.lax.dot_general(
        a, b, (((1,), (0,)), ((), ())),
        preferred_element_type=jnp.float32,
        precision=precision)


def _dott(a, w, precision=None):
    return jax.lax.dot_general(
        a, w, (((1,), (1,)), ((), ())),
        preferred_element_type=jnp.float32,
        precision=precision)


def _vqvae_kernel(x_ref, emb_ref,
                  w1_ref, b1_ref, w2_ref, b2_ref, w3_ref, b3_ref,
                  w4_ref, b4_ref, w5_ref, b5_ref, w6_ref, b6_ref,
                  recon_ref, q_ref, loss_ref):
    i = pl.program_id(0)

    h1 = jax.nn.relu(_dott(x_ref[...], w1_ref[...]) + b1_ref[...])
    h2 = jax.nn.relu(_dott(h1, w2_ref[...]) + b2_ref[...])
    f = jax.nn.relu(_dott(h2, w3_ref[...]) + b3_ref[...])

    emb = emb_ref[...]
    esq = _dott(jnp.ones((1, LAT), jnp.float32), emb * emb,
                precision=jax.lax.Precision.HIGHEST)
    sc = _dott(f, emb, precision=jax.lax.Precision.HIGHEST)
    score = esq - (sc + sc)

    iota = jax.lax.broadcasted_iota(jnp.int32, (TILE, K), 1)

    def first_min_idx(s):
        m = jnp.min(s, axis=1, keepdims=True)
        return jnp.min(jnp.where(s <= m, iota, K), axis=1)[:, None]

    def exact_dist(ec):
        t = f - ec
        t = t * t
        acc = None
        for a in range(4):
            v = t[:, 8 * a:8 * a + 8]
            v = v[:, 0:4] + v[:, 4:8]
            v = v[:, 0:2] + v[:, 2:4]
            v = v[:, 0:1] + v[:, 1:2]
            acc = v if acc is None else acc + v
        return acc

    cands = []
    s_cur = score
    for _ in range(3):
        i_c = first_min_idx(s_cur)
        oh = (iota == i_c).astype(jnp.float32)
        e_c = _dot(oh, emb, precision=jax.lax.Precision.HIGHEST)
        cands.append((exact_dist(e_c), i_c, e_c))
        s_cur = jnp.where(iota == i_c, jnp.inf, s_cur)

    d_w, i_w, e_w = cands[0]
    for d_c, i_c, e_c in cands[1:]:
        take = (d_c < d_w) | ((d_c == d_w) & (i_c < i_w))
        d_w = jnp.where(take, d_c, d_w)
        i_w = jnp.where(take, i_c, i_w)
        e_w = jnp.where(take, e_c, e_w)
    q = e_w

    dq = q - f
    part = jnp.sum(dq * dq)
    q = f + dq

    @pl.when(i == 0)
    def _():
        loss_ref[0, 0] = 0.0
    loss_ref[0, 0] += part
    @pl.when(i == GRID - 1)
    def _():
        loss_ref[0, 0] *= 1.25 / (B * LAT)

    d1 = jax.nn.relu(_dott(q, w4_ref[...]) + b4_ref[...])
    d2 = jax.nn.relu(_dott(d1, w5_ref[...]) + b5_ref[...])
    recon_ref[...] = jax.nn.sigmoid(_dott(d2, w6_ref[...]) + b6_ref[...])
    q_ref[...] = q


@functools.partial(jax.jit, static_argnames=("interpret",))
def kernel(x, emb, enc1_w, enc1_b, enc2_w, enc2_b, enc3_w, enc3_b,
           dec1_w, dec1_b, dec2_w, dec2_b, dec3_w, dec3_b, interpret=False):
    rep = lambda shape: pl.BlockSpec(shape, lambda i: (0,) * len(shape))
    w_specs = []
    ws = []
    for w, b in ((enc1_w, enc1_b), (enc2_w, enc2_b), (enc3_w, enc3_b),
                 (dec1_w, dec1_b), (dec2_w, dec2_b), (dec3_w, dec3_b)):
        ws += [w, b.reshape(1, -1)]
        w_specs += [rep(w.shape), rep((1, b.shape[0]))]

    recon, q, loss = pl.pallas_call(
        _vqvae_kernel,
        grid=(GRID,),
        in_specs=[pl.BlockSpec((TILE, CIN), lambda i: (i, 0)),
                  rep((K, LAT))] + w_specs,
        out_specs=[pl.BlockSpec((TILE, CIN), lambda i: (i, 0)),
                   pl.BlockSpec((TILE, LAT), lambda i: (i, 0)),
                   pl.BlockSpec(memory_space=pltpu.SMEM)],
        out_shape=[jax.ShapeDtypeStruct((B, CIN), jnp.float32),
                   jax.ShapeDtypeStruct((B, LAT), jnp.float32),
                   jax.ShapeDtypeStruct((1, 1), jnp.float32)],
        interpret=interpret,
    )(x, emb, *ws)

    return (recon, loss[0, 0], q)
